# baseline (speedup 1.0000x reference)
import functools

import jax
import jax.numpy as jnp
from jax import lax
from jax.experimental import pallas as pl
from jax.experimental.pallas import tpu as pltpu
from jax.experimental.pallas import tpu_sc as plsc

L = 16
ROWS = 128
N = 32768
NV = N // L
RANK = 513
NWORKERS = 32
ROWS_PER_W = ROWS // NWORKERS
HIST = 256
MIN_I32 = -2147483648


def _select_digit(hist_ref, lane, r):
    found = jnp.int32(0)
    bstar = jnp.int32(0)
    cab = jnp.int32(0)
    carry = jnp.int32(0)
    r_v = jnp.full((L,), r, jnp.int32)
    zeros = jnp.zeros((L,), jnp.int32)
    for c in range(15, -1, -1):
        tot = jnp.zeros((L,), jnp.int32)
        for l in range(16):
            tot = tot + hist_ref[pl.ds(l * HIST + c * L, L)]
            hist_ref[pl.ds(l * HIST + c * L, L)] = zeros
        suf = lax.rev(jnp.cumsum(lax.rev(tot, (0,))), (0,))
        T = suf + carry
        m = T >= r_v
        cnt = jnp.max(plsc.all_reduce_population_count(m))
        has = cnt > 0
        j = cnt - 1
        above = jnp.where(lane == j, T - tot, 0)
        c_here = jnp.max(above)
        take = jnp.logical_and(has, found == 0)
        bstar = jnp.where(take, jnp.int32(16 * c) + j, bstar)
        cab = jnp.where(take, c_here, cab)
        found = jnp.where(has, jnp.int32(1), found)
        carry = carry + jnp.sum(tot)
    return bstar, cab


def _body(in_hbm, out_hbm, x_v, u_v, hist_v):
    cid = lax.axis_index("c")
    sid = lax.axis_index("s")
    wid = sid * 2 + cid
    lane = lax.iota(jnp.int32, L)
    lane_off = lane * HIST
    ones = jnp.ones((L,), jnp.int32)
    zeros = jnp.zeros((L,), jnp.int32)
    sign_v = jnp.full((L,), MIN_I32, jnp.int32)

    @plsc.parallel_loop(0, HIST * 16 // L, unroll=8)
    def zero_hist(j):
        hist_v[pl.ds(j * L, L)] = zeros

    def do_row(i, _):
        row = wid * ROWS_PER_W + i
        pltpu.sync_copy(in_hbm.at[row], x_v)

        @plsc.parallel_loop(0, NV, unroll=16)
        def p3(j):
            x = x_v[pl.ds(j * L, L)]
            v = plsc.bitcast(x, jnp.int32)
            s = lax.shift_right_arithmetic(v, 31)
            u = lax.bitwise_xor(v, lax.bitwise_or(s, sign_v))
            u_v[pl.ds(j * L, L)] = u
            d = lax.shift_right_logical(u, 24)
            plsc.addupdate_scatter(hist_v, [lane_off + d], ones)
        b, cab = _select_digit(hist_v, lane, jnp.int32(RANK))
        prefix0 = b
        r0 = jnp.int32(RANK) - cab

        def digit_pass(p, pr):
            prefix, r = pr
            hi = 24 - 8 * p
            lo = 16 - 8 * p
            pref_v = jnp.full((L,), prefix, jnp.int32)

            @plsc.parallel_loop(0, NV, unroll=16)
            def hbody(j):
                u = u_v[pl.ds(j * L, L)]
                match = lax.shift_right_logical(u, hi) == pref_v
                d = lax.bitwise_and(lax.shift_right_logical(u, lo), 255)
                plsc.addupdate_scatter(hist_v, [lane_off + d], ones,
                                       mask=match)
            b2, cab2 = _select_digit(hist_v, lane, r)
            return (lax.shift_left(prefix, 8) | b2, r - cab2)

        prefix, _r = lax.fori_loop(0, 3, digit_pass, (prefix0, r0))

        ut_v = jnp.full((L,), prefix, jnp.int32)
        xmask = lax.bitwise_or(
            lax.bitwise_not(lax.shift_right_arithmetic(ut_v, 31)), sign_v)
        t_v = plsc.bitcast(lax.bitwise_xor(ut_v, xmask), jnp.float32)

        @plsc.parallel_loop(0, NV, unroll=16)
        def mbody(j):
            x = x_v[pl.ds(j * L, L)]
            x_v[pl.ds(j * L, L)] = jnp.where(x > t_v, x, 0.0)
        pltpu.sync_copy(x_v, out_hbm.at[row])
        return 0

    lax.fori_loop(0, ROWS_PER_W, do_row, 0)


@jax.jit
def _ksparse(inputs):
    mesh = plsc.VectorSubcoreMesh(core_axis_name="c", subcore_axis_name="s")
    f = functools.partial(
        pl.kernel,
        mesh=mesh,
        out_type=jax.ShapeDtypeStruct((ROWS, N), jnp.float32),
        compiler_params=pltpu.CompilerParams(needs_layout_passes=False),
        scratch_types=[
            pltpu.VMEM((N,), jnp.float32),
            pltpu.VMEM((N,), jnp.int32),
            pltpu.VMEM((16 * HIST,), jnp.int32),
        ],
    )(_body)
    return f(inputs)


def kernel(inputs):
    return _ksparse(inputs)


# speedup vs baseline: 1.2317x; 1.2317x over previous
import functools

import jax
import jax.numpy as jnp
from jax import lax
from jax.experimental import pallas as pl
from jax.experimental.pallas import tpu as pltpu
from jax.experimental.pallas import tpu_sc as plsc

L = 16
ROWS = 128
N = 32768
NV = N // L
RANK = 513
NWORKERS = 32
ROWS_PER_W = ROWS // NWORKERS
HIST = 256
HSTRIDE = 257
MIN_I32 = -2147483648


def _select_digit(hist_ref, lane, r):
    found = jnp.int32(0)
    bstar = jnp.int32(0)
    cab = jnp.int32(0)
    carry = jnp.int32(0)
    r_v = jnp.full((L,), r, jnp.int32)
    zeros = jnp.zeros((L,), jnp.int32)
    for c in range(15, -1, -1):
        tot = jnp.zeros((L,), jnp.int32)
        for l in range(16):
            tot = tot + hist_ref[pl.ds(l * HSTRIDE + c * L, L)]
            hist_ref[pl.ds(l * HSTRIDE + c * L, L)] = zeros
        suf = lax.rev(jnp.cumsum(lax.rev(tot, (0,))), (0,))
        T = suf + carry
        m = T >= r_v
        cnt = jnp.max(plsc.all_reduce_population_count(m))
        has = cnt > 0
        j = cnt - 1
        above = jnp.where(lane == j, T - tot, 0)
        c_here = jnp.max(above)
        take = jnp.logical_and(has, found == 0)
        bstar = jnp.where(take, jnp.int32(16 * c) + j, bstar)
        cab = jnp.where(take, c_here, cab)
        found = jnp.where(has, jnp.int32(1), found)
        carry = carry + jnp.sum(tot)
    return bstar, cab


def _body(in_hbm, out_hbm, x_v, u_v, hist_v):
    cid = lax.axis_index("c")
    sid = lax.axis_index("s")
    wid = sid * 2 + cid
    lane = lax.iota(jnp.int32, L)
    lane_off = lane * HSTRIDE
    ones = jnp.ones((L,), jnp.int32)
    zeros = jnp.zeros((L,), jnp.int32)
    sign_v = jnp.full((L,), MIN_I32, jnp.int32)

    @plsc.parallel_loop(0, HSTRIDE * 16 // L, unroll=8)
    def zero_hist(j):
        hist_v[pl.ds(j * L, L)] = zeros

    def do_row(i, _):
        row = wid * ROWS_PER_W + i
        pltpu.sync_copy(in_hbm.at[row], x_v)

        @plsc.parallel_loop(0, NV, unroll=16)
        def p3(j):
            x = x_v[pl.ds(j * L, L)]
            v = plsc.bitcast(x, jnp.int32)
            s = lax.shift_right_arithmetic(v, 31)
            u = lax.bitwise_xor(v, lax.bitwise_or(s, sign_v))
            u_v[pl.ds(j * L, L)] = u
            d = lax.shift_right_logical(u, 24)
            plsc.addupdate_scatter(hist_v, [lane_off + d], ones)
        b, cab = _select_digit(hist_v, lane, jnp.int32(RANK))
        prefix0 = b
        r0 = jnp.int32(RANK) - cab

        def digit_pass(p, pr):
            prefix, r = pr
            hi = 24 - 8 * p
            lo = 16 - 8 * p
            pref_v = jnp.full((L,), prefix, jnp.int32)

            @plsc.parallel_loop(0, NV, unroll=16)
            def hbody(j):
                u = u_v[pl.ds(j * L, L)]
                match = lax.shift_right_logical(u, hi) == pref_v
                d = lax.bitwise_and(lax.shift_right_logical(u, lo), 255)
                plsc.addupdate_scatter(hist_v, [lane_off + d], ones,
                                       mask=match)
            b2, cab2 = _select_digit(hist_v, lane, r)
            return (lax.shift_left(prefix, 8) | b2, r - cab2)

        prefix, _r = lax.fori_loop(0, 3, digit_pass, (prefix0, r0))

        ut_v = jnp.full((L,), prefix, jnp.int32)
        xmask = lax.bitwise_or(
            lax.bitwise_not(lax.shift_right_arithmetic(ut_v, 31)), sign_v)
        t_v = plsc.bitcast(lax.bitwise_xor(ut_v, xmask), jnp.float32)

        @plsc.parallel_loop(0, NV, unroll=16)
        def mbody(j):
            x = x_v[pl.ds(j * L, L)]
            x_v[pl.ds(j * L, L)] = jnp.where(x > t_v, x, 0.0)
        pltpu.sync_copy(x_v, out_hbm.at[row])
        return 0

    lax.fori_loop(0, ROWS_PER_W, do_row, 0)


@jax.jit
def _ksparse(inputs):
    mesh = plsc.VectorSubcoreMesh(core_axis_name="c", subcore_axis_name="s")
    f = functools.partial(
        pl.kernel,
        mesh=mesh,
        out_type=jax.ShapeDtypeStruct((ROWS, N), jnp.float32),
        compiler_params=pltpu.CompilerParams(needs_layout_passes=False),
        scratch_types=[
            pltpu.VMEM((N,), jnp.float32),
            pltpu.VMEM((N,), jnp.int32),
            pltpu.VMEM((16 * HSTRIDE,), jnp.int32),
        ],
    )(_body)
    return f(inputs)


def kernel(inputs):
    return _ksparse(inputs)


# speedup vs baseline: 1.3376x; 1.0860x over previous
import functools

import jax
import jax.numpy as jnp
from jax import lax
from jax.experimental import pallas as pl
from jax.experimental.pallas import tpu as pltpu
from jax.experimental.pallas import tpu_sc as plsc

L = 16
ROWS = 128
N = 32768
NV = N // L
RANK = 513
NWORKERS = 32
ROWS_PER_W = ROWS // NWORKERS
HIST = 256
HSTRIDE = 257
MIN_I32 = -2147483648


def _select_digit(hist_ref, lane, r):
    found = jnp.int32(0)
    bstar = jnp.int32(0)
    cab = jnp.int32(0)
    carry = jnp.int32(0)
    r_v = jnp.full((L,), r, jnp.int32)
    zeros = jnp.zeros((L,), jnp.int32)
    for c in range(15, -1, -1):
        tot = jnp.zeros((L,), jnp.int32)
        for l in range(16):
            tot = tot + hist_ref[pl.ds(l * HSTRIDE + c * L, L)]
            hist_ref[pl.ds(l * HSTRIDE + c * L, L)] = zeros
        suf = lax.rev(jnp.cumsum(lax.rev(tot, (0,))), (0,))
        T = suf + carry
        m = T >= r_v
        cnt = jnp.max(plsc.all_reduce_population_count(m))
        has = cnt > 0
        j = cnt - 1
        above = jnp.where(lane == j, T - tot, 0)
        c_here = jnp.max(above)
        take = jnp.logical_and(has, found == 0)
        bstar = jnp.where(take, jnp.int32(16 * c) + j, bstar)
        cab = jnp.where(take, c_here, cab)
        found = jnp.where(has, jnp.int32(1), found)
        carry = carry + jnp.sum(tot)
    return bstar, cab


def _body(in_hbm, out_hbm, x_v, u_v, c_v, hist_v):
    cid = lax.axis_index("c")
    sid = lax.axis_index("s")
    wid = sid * 2 + cid
    lane = lax.iota(jnp.int32, L)
    lane_off = lane * HSTRIDE
    ones = jnp.ones((L,), jnp.int32)
    zeros = jnp.zeros((L,), jnp.int32)
    sign_v = jnp.full((L,), MIN_I32, jnp.int32)

    @plsc.parallel_loop(0, HSTRIDE * 16 // L, unroll=8)
    def zero_hist(j):
        hist_v[pl.ds(j * L, L)] = zeros

    def do_row(i, _):
        row = wid * ROWS_PER_W + i
        pltpu.sync_copy(in_hbm.at[row], x_v)

        @plsc.parallel_loop(0, NV, unroll=16)
        def p3(j):
            x = x_v[pl.ds(j * L, L)]
            v = plsc.bitcast(x, jnp.int32)
            s = lax.shift_right_arithmetic(v, 31)
            u = lax.bitwise_xor(v, lax.bitwise_or(s, sign_v))
            u_v[pl.ds(j * L, L)] = u
            d = lax.shift_right_logical(u, 24)
            plsc.addupdate_scatter(hist_v, [lane_off + d], ones)
        b, cab = _select_digit(hist_v, lane, jnp.int32(RANK))
        prefix = b
        r = jnp.int32(RANK) - cab

        pref3_v = jnp.full((L,), prefix, jnp.int32)

        @plsc.parallel_loop(0, NV, unroll=8, carry=jnp.zeros((L,), jnp.int32))
        def p2(j, off_v):
            u = u_v[pl.ds(j * L, L)]
            match = lax.shift_right_logical(u, 24) == pref3_v
            d = lax.bitwise_and(lax.shift_right_logical(u, 16), 255)
            plsc.addupdate_scatter(hist_v, [lane_off + d], ones, mask=match)
            mi = match.astype(jnp.int32)
            pos = off_v + jnp.cumsum(mi) - ones
            plsc.store_scatter(c_v, [pos], u, mask=match)
            return off_v + plsc.all_reduce_population_count(match)

        cnt_v = p2
        b, cab = _select_digit(hist_v, lane, r)
        prefix = lax.shift_left(prefix, 8) | b
        r = r - cab
        nv2 = (jnp.max(cnt_v) + (L - 1)) // L

        pref2_v = jnp.full((L,), prefix, jnp.int32)

        @plsc.parallel_loop(0, nv2, unroll=4)
        def p1(j):
            u = c_v[pl.ds(j * L, L)]
            valid = (j * L + lane) < cnt_v
            match = jnp.logical_and(
                lax.shift_right_logical(u, 16) == pref2_v, valid)
            d = lax.bitwise_and(lax.shift_right_logical(u, 8), 255)
            plsc.addupdate_scatter(hist_v, [lane_off + d], ones, mask=match)

        b, cab = _select_digit(hist_v, lane, r)
        prefix = lax.shift_left(prefix, 8) | b
        r = r - cab

        pref1_v = jnp.full((L,), prefix, jnp.int32)

        @plsc.parallel_loop(0, nv2, unroll=4)
        def p0(j):
            u = c_v[pl.ds(j * L, L)]
            valid = (j * L + lane) < cnt_v
            match = jnp.logical_and(
                lax.shift_right_logical(u, 8) == pref1_v, valid)
            d = lax.bitwise_and(u, 255)
            plsc.addupdate_scatter(hist_v, [lane_off + d], ones, mask=match)

        b, _cab = _select_digit(hist_v, lane, r)
        prefix = lax.shift_left(prefix, 8) | b

        ut_v = jnp.full((L,), prefix, jnp.int32)
        xmask = lax.bitwise_or(
            lax.bitwise_not(lax.shift_right_arithmetic(ut_v, 31)), sign_v)
        t_v = plsc.bitcast(lax.bitwise_xor(ut_v, xmask), jnp.float32)

        @plsc.parallel_loop(0, NV, unroll=16)
        def mbody(j):
            x = x_v[pl.ds(j * L, L)]
            x_v[pl.ds(j * L, L)] = jnp.where(x > t_v, x, 0.0)
        pltpu.sync_copy(x_v, out_hbm.at[row])
        return 0

    lax.fori_loop(0, ROWS_PER_W, do_row, 0)


@jax.jit
def _ksparse(inputs):
    mesh = plsc.VectorSubcoreMesh(core_axis_name="c", subcore_axis_name="s")
    f = functools.partial(
        pl.kernel,
        mesh=mesh,
        out_type=jax.ShapeDtypeStruct((ROWS, N), jnp.float32),
        compiler_params=pltpu.CompilerParams(needs_layout_passes=False),
        scratch_types=[
            pltpu.VMEM((N,), jnp.float32),
            pltpu.VMEM((N,), jnp.int32),
            pltpu.VMEM((N,), jnp.int32),
            pltpu.VMEM((16 * HSTRIDE,), jnp.int32),
        ],
    )(_body)
    return f(inputs)


def kernel(inputs):
    return _ksparse(inputs)
